# baseline (speedup 1.0000x reference)
import jax
import jax.numpy as jnp
from jax.experimental import pallas as pl


def _gcn_conv(x, edge_index, W, b):
    N = x.shape[0]
    loops = jnp.arange(N, dtype=edge_index.dtype)
    src = jnp.concatenate([edge_index[0], loops])
    dst = jnp.concatenate([edge_index[1], loops])
    deg = jnp.zeros((N,), x.dtype).at[dst].add(1.0)
    dinv = jnp.where(deg > 0, 1.0 / jnp.sqrt(deg), 0.0)
    norm = dinv[src] * dinv[dst]
    h = x @ W
    msg = h[src] * norm[:, None]
    out = jnp.zeros((N, W.shape[1]), x.dtype).at[dst].add(msg)
    return out + b


def _bias_kernel(x_ref, b_ref, o_ref):
    o_ref[...] = x_ref[...] + b_ref[...]


def kernel(x, edge_index, batch, W1, b1, W2, b2, W3, b3, Wfc, bfc):
    h = jax.nn.relu(_gcn_conv(x, edge_index, W1, b1))
    h = jax.nn.relu(_gcn_conv(h, edge_index, W2, b2))
    h = jax.nn.relu(_gcn_conv(h, edge_index, W3, b3))
    sums = jax.ops.segment_sum(h, batch, num_segments=256)
    counts = jax.ops.segment_sum(jnp.ones((h.shape[0],), h.dtype), batch,
                                 num_segments=256)
    pooled = sums / jnp.maximum(counts, 1.0)[:, None]
    logits = pooled @ Wfc
    return pl.pallas_call(
        _bias_kernel,
        out_shape=jax.ShapeDtypeStruct(logits.shape, logits.dtype),
    )(logits, jnp.broadcast_to(bfc, logits.shape))


# speedup vs baseline: 9.4353x; 9.4353x over previous
import functools

import jax
import jax.numpy as jnp
from jax import lax
from jax.experimental import pallas as pl
from jax.experimental.pallas import tpu as pltpu
from jax.experimental.pallas import tpu_sc as plsc

NC = 2
NS = 16
LN = 16
KROWS = 8
RB = 1024
HIGH = jax.lax.Precision.HIGHEST


def _sc_scatter_kernel(num_slices, nbig, np_, nacc, er):
  stripe = np_ // NS

  def body(srcr, dstr, zeros, *rest):
    g_refs = rest[:num_slices]
    out = rest[num_slices]
    sbuf, dbuf, rows, acc, gsem, ssem = rest[num_slices + 1:]
    c = lax.axis_index("c")
    s_id = lax.axis_index("s")
    rowbase = (c * NS + s_id) * (nbig * KROWS)

    for sl in range(num_slices):
      pltpu.sync_copy(zeros, acc.at[pl.ds(s_id * stripe, stripe)])
      plsc.subcore_barrier()

      def step(k, carry):
        r0 = rowbase + k * KROWS
        pltpu.sync_copy(srcr.at[pl.ds(r0, KROWS)], sbuf)
        pltpu.sync_copy(dstr.at[pl.ds(r0, KROWS)], dbuf)
        gs = [pltpu.async_copy(g_refs[sl].at[sbuf.at[j]], rows.at[j], gsem)
              for j in range(KROWS)]
        for d in gs:
          d.wait()
        ss = [pltpu.async_copy(rows.at[j], acc.at[dbuf.at[j]], ssem, add=True)
              for j in range(KROWS)]
        for d in ss:
          d.wait()
        return carry

      lax.fori_loop(0, nbig, step, 0)
      plsc.subcore_barrier()
      pltpu.sync_copy(acc.at[pl.ds(s_id * stripe, stripe)],
                      out.at[c, sl, pl.ds(s_id * stripe, stripe)])
      plsc.subcore_barrier()

  return pl.kernel(
      body,
      out_type=jax.ShapeDtypeStruct((NC, num_slices, np_, LN), jnp.float32),
      mesh=plsc.VectorSubcoreMesh(core_axis_name="c", subcore_axis_name="s"),
      compiler_params=pltpu.CompilerParams(use_tc_tiling_on_sc=False),
      scratch_types=[
          pltpu.VMEM((KROWS, 128), jnp.int32),
          pltpu.VMEM((KROWS, 128), jnp.int32),
          pltpu.VMEM((KROWS, 128, LN), jnp.float32),
          pltpu.VMEM_SHARED((nacc, LN), jnp.float32),
          pltpu.SemaphoreType.DMA,
          pltpu.SemaphoreType.DMA,
      ],
  )


def _sc_degree_kernel(nbig, np_, nacc, er):
  stripe = np_ // NS

  def body(dstr, zeros, ones, out, dbuf, rows, acc, ssem):
    c = lax.axis_index("c")
    s_id = lax.axis_index("s")
    rowbase = (c * NS + s_id) * (nbig * KROWS)
    pltpu.sync_copy(ones, rows)
    pltpu.sync_copy(zeros, acc.at[pl.ds(s_id * stripe, stripe)])
    plsc.subcore_barrier()

    def step(k, carry):
      r0 = rowbase + k * KROWS
      pltpu.sync_copy(dstr.at[pl.ds(r0, KROWS)], dbuf)
      ss = [pltpu.async_copy(rows.at[j], acc.at[dbuf.at[j]], ssem, add=True)
            for j in range(KROWS)]
      for d in ss:
        d.wait()
      return carry

    lax.fori_loop(0, nbig, step, 0)
    plsc.subcore_barrier()
    pltpu.sync_copy(acc.at[pl.ds(s_id * stripe, stripe)],
                    out.at[c, 0, pl.ds(s_id * stripe, stripe)])

  return pl.kernel(
      body,
      out_type=jax.ShapeDtypeStruct((NC, 1, np_, LN), jnp.float32),
      mesh=plsc.VectorSubcoreMesh(core_axis_name="c", subcore_axis_name="s"),
      compiler_params=pltpu.CompilerParams(use_tc_tiling_on_sc=False),
      scratch_types=[
          pltpu.VMEM((KROWS, 128), jnp.int32),
          pltpu.VMEM((KROWS, 128, LN), jnp.float32),
          pltpu.VMEM_SHARED((nacc, LN), jnp.float32),
          pltpu.SemaphoreType.DMA,
      ],
  )


def _tc_norm_kernel(degp_ref, xp_ref, dinv_ref, g1_ref):
  deg = degp_ref[0, 0] + degp_ref[1, 0] + 1.0
  dinv = lax.rsqrt(deg)
  dinv_ref[...] = dinv
  g1_ref[...] = dinv * xp_ref[...]


def _tc_layer_kernel(s_in, s_out, last, acc_ref, dinv_ref, w_ref, b_ref,
                     *rest):
  g_refs = rest[:s_in]
  o_refs = rest[s_in:]
  dinv = dinv_ref[...]
  a = jnp.concatenate(
      [dinv * (acc_ref[0, s] + acc_ref[1, s] + g_refs[s][...])
       for s in range(s_in)], axis=1)
  h = jnp.maximum(
      jnp.dot(a, w_ref[...], precision=HIGH,
              preferred_element_type=jnp.float32) + b_ref[...], 0.0)
  if last:
    n = h.shape[0]
    aug = jnp.concatenate(
        [h, jnp.ones((n, 1), jnp.float32), jnp.zeros((n, 15), jnp.float32)],
        axis=1)
    o_refs[0][...] = aug
  else:
    scale = dinv[:, 0:1]
    for s in range(s_out):
      o_refs[s][...] = scale * h[:, s * LN:(s + 1) * LN]


def _tc_pool_kernel(batch_ref, aug_ref, out_ref):
  b = batch_ref[...]
  gid = lax.broadcasted_iota(jnp.int32, (b.shape[0], 256), 1)
  onehot = (b == gid).astype(jnp.float32)
  part = lax.dot_general(onehot, aug_ref[...], (((0,), (0,)), ((), ())),
                         precision=HIGH, preferred_element_type=jnp.float32)

  @pl.when(pl.program_id(0) == 0)
  def _():
    out_ref[...] = jnp.zeros_like(out_ref)

  out_ref[...] += part


def _tc_fc_kernel(pool_ref, w_ref, b_ref, out_ref):
  pool = pool_ref[...]
  sums = pool[:, :256]
  cnt = pool[:, 256:257]
  pooled = sums / jnp.maximum(cnt, 1.0)
  out_ref[...] = jnp.dot(pooled, w_ref[...], precision=HIGH,
                         preferred_element_type=jnp.float32) + b_ref[...]


def kernel(x, edge_index, batch, W1, b1, W2, b2, W3, b3, Wfc, bfc):
  n = x.shape[0]
  e = edge_index.shape[1]
  f32 = jnp.float32
  i32 = jnp.int32

  np_ = -(-n // (NS * NC * 128)) * (NS * NC * 128)
  nacc = np_ + 8
  ep = -(-e // (NS * NC * KROWS * 128)) * (NS * NC * KROWS * 128)
  er = ep // 128
  nbig = er // (NS * NC) // KROWS
  ngrid = np_ // RB

  pad_e = ep - e
  srcr = jnp.concatenate([edge_index[0], jnp.zeros((pad_e,), i32)])
  srcr = srcr.reshape(er, 128)
  dstr = jnp.concatenate([edge_index[1], jnp.full((pad_e,), np_, i32)])
  dstr = dstr.reshape(er, 128)
  batch2d = jnp.concatenate([batch, jnp.full((np_ - n,), 256, i32)])
  batch2d = batch2d.reshape(np_, 1)
  xp = jnp.zeros((np_, LN), f32).at[:n, :3].set(x)
  zeros_c = jnp.zeros((np_ // NS, LN), f32)
  ones_c = jnp.ones((KROWS, 128, LN), f32)
  w1p = jnp.zeros((LN, 64), f32).at[:3].set(W1)
  wfcp = jnp.zeros((256, LN), f32).at[:, :10].set(Wfc)
  bfcp = jnp.zeros((1, LN), f32).at[:, :10].set(bfc)

  def nblk(shape):
    nd = len(shape)
    return pl.BlockSpec(shape, lambda i, _nd=nd: (0,) * (_nd - 2) + (i, 0))

  def wblk(shape):
    nd = len(shape)
    return pl.BlockSpec(shape, lambda i, _nd=nd: (0,) * _nd)

  degp = _sc_degree_kernel(nbig, np_, nacc, er)(dstr, zeros_c, ones_c)

  dinv, g1 = pl.pallas_call(
      _tc_norm_kernel,
      grid=(ngrid,),
      in_specs=[nblk((NC, 1, RB, LN)), nblk((RB, LN))],
      out_specs=[nblk((RB, LN)), nblk((RB, LN))],
      out_shape=[jax.ShapeDtypeStruct((np_, LN), f32)] * 2,
  )(degp, xp)

  def layer(g_slices, w, b, s_out, last):
    s_in = len(g_slices)
    acc = _sc_scatter_kernel(s_in, nbig, np_, nacc, er)(
        srcr, dstr, zeros_c, *g_slices)
    fw = w.shape[1]
    if last:
      out_specs = [nblk((RB, 272))]
      out_shape = [jax.ShapeDtypeStruct((np_, 272), f32)]
    else:
      out_specs = [nblk((RB, LN))] * s_out
      out_shape = [jax.ShapeDtypeStruct((np_, LN), f32)] * s_out
    outs = pl.pallas_call(
        functools.partial(_tc_layer_kernel, s_in, s_out, last),
        grid=(ngrid,),
        in_specs=[nblk((NC, s_in, RB, LN)), nblk((RB, LN)),
                  wblk(w.shape), wblk((1, fw))]
                 + [nblk((RB, LN))] * s_in,
        out_specs=out_specs,
        out_shape=out_shape,
    )(acc, dinv, w, b.reshape(1, fw), *g_slices)
    return outs

  g2 = layer([g1], w1p, b1, 4, False)
  g3 = layer(list(g2), W2, b2, 8, False)
  aug = layer(list(g3), W3, b3, 1, True)[0]

  pool = pl.pallas_call(
      _tc_pool_kernel,
      grid=(ngrid,),
      in_specs=[nblk((RB, 1)), nblk((RB, 272))],
      out_specs=wblk((256, 272)),
      out_shape=jax.ShapeDtypeStruct((256, 272), f32),
  )(batch2d, aug)

  logits = pl.pallas_call(
      _tc_fc_kernel,
      grid=(1,),
      in_specs=[wblk((256, 272)), wblk((256, LN)), wblk((1, LN))],
      out_specs=wblk((256, LN)),
      out_shape=jax.ShapeDtypeStruct((256, LN), f32),
  )(pool, wfcp, bfcp)

  return logits[:, :10]
